# Initial kernel scaffold; baseline (speedup 1.0000x reference)
import functools

import jax
import jax.numpy as jnp
from jax import lax
from jax.experimental import pallas as pl
from jax.experimental.pallas import tpu as pltpu
from jax.experimental.pallas import tpu_sc as plsc

NC = 2
NS = 16
NW = NC * NS

B_EDGE = 80


def _sc_degree(dst3, ones_rows, zero16):
    nw, ch, bb = dst3.shape
    rpt, _ = zero16.shape
    n = rpt * NS
    mesh = plsc.VectorSubcoreMesh(core_axis_name="c", subcore_axis_name="s")

    @functools.partial(
        pl.kernel,
        out_type=jax.ShapeDtypeStruct((NC, n, 16), jnp.float32),
        mesh=mesh,
        scratch_types=[
            pltpu.VMEM((ch, bb), jnp.int32),
            pltpu.VMEM((bb, 16), jnp.float32),
            pltpu.VMEM_SHARED((n, 16), jnp.float32),
        ],
    )
    def k(dst_hbm, ones_hbm, zero_hbm, out_hbm, slab, ones_v, acc):
        c = lax.axis_index("c")
        s = lax.axis_index("s")
        w = s * NC + c
        pltpu.sync_copy(dst_hbm.at[w], slab)
        pltpu.sync_copy(ones_hbm, ones_v)
        pltpu.sync_copy(zero_hbm, acc.at[pl.ds(s * rpt, rpt)])
        plsc.subcore_barrier()

        def chunk(j, carry):
            pltpu.sync_copy(ones_v, acc.at[slab.at[j]], add=True)
            return carry

        lax.fori_loop(0, ch, chunk, 0)
        plsc.subcore_barrier()
        pltpu.sync_copy(acc.at[pl.ds(s * rpt, rpt)],
                        out_hbm.at[c, pl.ds(s * rpt, rpt)])

    return k(dst3, ones_rows, zero16)


def _sc_aggregate(zd, src3, dst3, zero_rows):
    n, f = zd.shape
    nw, ch, bb = src3.shape
    rpt = n // NS
    mesh = plsc.VectorSubcoreMesh(core_axis_name="c", subcore_axis_name="s")

    @functools.partial(
        pl.kernel,
        out_type=jax.ShapeDtypeStruct((NC, n, f), jnp.float32),
        mesh=mesh,
        scratch_types=[
            pltpu.VMEM((ch, bb), jnp.int32),
            pltpu.VMEM((ch, bb), jnp.int32),
            pltpu.VMEM((bb, f), jnp.float32),
            pltpu.VMEM_SHARED((n, f), jnp.float32),
            pltpu.SemaphoreType.DMA,
        ],
    )
    def k(zd_hbm, src_hbm, dst_hbm, zero_hbm, out_hbm,
          sslab, dslab, rows, acc, sem):
        c = lax.axis_index("c")
        s = lax.axis_index("s")
        w = s * NC + c
        pltpu.sync_copy(src_hbm.at[w], sslab)
        pltpu.sync_copy(dst_hbm.at[w], dslab)
        pltpu.sync_copy(zero_hbm, acc.at[pl.ds(s * rpt, rpt)])
        plsc.subcore_barrier()

        def chunk(j, carry):
            pltpu.async_copy(zd_hbm.at[sslab.at[j]], rows, sem).wait()
            pltpu.sync_copy(rows, acc.at[dslab.at[j]], add=True)
            return carry

        lax.fori_loop(0, ch, chunk, 0)
        plsc.subcore_barrier()
        pltpu.sync_copy(acc.at[pl.ds(s * rpt, rpt)],
                        out_hbm.at[c, pl.ds(s * rpt, rpt)])

    return k(zd, src3, dst3, zero_rows)


def _deg_to_d(deg_ref):
    dg = deg_ref[0, :, 0:1] + deg_ref[1, :, 0:1] + 1.0
    return lax.rsqrt(dg)


def _tc_layer1(x, w1, deg, blk):
    n, f = x.shape
    h = w1.shape[1]
    grid = (n // blk,)

    def body(x_ref, w_ref, deg_ref, o_ref):
        d = _deg_to_d(deg_ref)
        z = jnp.dot(x_ref[...], w_ref[...], preferred_element_type=jnp.float32)
        o_ref[...] = z * d

    return pl.pallas_call(
        body,
        grid=grid,
        in_specs=[
            pl.BlockSpec((blk, f), lambda i: (i, 0)),
            pl.BlockSpec((f, h), lambda i: (0, 0)),
            pl.BlockSpec((NC, blk, 16), lambda i: (0, i, 0)),
        ],
        out_specs=pl.BlockSpec((blk, h), lambda i: (i, 0)),
        out_shape=jax.ShapeDtypeStruct((n, h), jnp.float32),
    )(x, w1, deg)


def _tc_layer(agg, zd_prev, deg, b_prev, w, blk):
    n, f = zd_prev.shape
    h = w.shape[1]
    grid = (n // blk,)

    def body(agg_ref, zd_ref, deg_ref, b_ref, w_ref, o_ref):
        d = _deg_to_d(deg_ref)
        pre = d * (agg_ref[0] + agg_ref[1] + zd_ref[...]) + b_ref[...]
        a = jnp.maximum(pre, 0.0)
        z = jnp.dot(a, w_ref[...], preferred_element_type=jnp.float32)
        o_ref[...] = z * d

    return pl.pallas_call(
        body,
        grid=grid,
        in_specs=[
            pl.BlockSpec((NC, blk, f), lambda i: (0, i, 0)),
            pl.BlockSpec((blk, f), lambda i: (i, 0)),
            pl.BlockSpec((NC, blk, 16), lambda i: (0, i, 0)),
            pl.BlockSpec((1, f), lambda i: (0, 0)),
            pl.BlockSpec((f, h), lambda i: (0, 0)),
        ],
        out_specs=pl.BlockSpec((blk, h), lambda i: (i, 0)),
        out_shape=jax.ShapeDtypeStruct((n, h), jnp.float32),
    )(agg, zd_prev, deg, b_prev, w)


def _tc_pool(agg, zd3, deg, b3, batch3, wl, bl, num_graphs, blk):
    n, f = zd3.shape
    outdim = wl.shape[1]
    g = num_graphs
    nblk = n // blk

    def body(agg_ref, zd_ref, deg_ref, b_ref, batch_ref, wl_ref, bl_ref,
             out_ref, emb_ref, s_ref, c_ref):
        i = pl.program_id(0)
        d = _deg_to_d(deg_ref)
        h3 = d * (agg_ref[0] + agg_ref[1] + zd_ref[...]) + b_ref[...]
        bvec = batch_ref[0, 0, :]
        gid = lax.broadcasted_iota(jnp.int32, (g, blk), 0)
        oht = (gid == bvec[None, :]).astype(jnp.float32)

        @pl.when(i == 0)
        def _():
            s_ref[...] = jnp.zeros_like(s_ref)
            c_ref[...] = jnp.zeros_like(c_ref)

        s_ref[...] += jnp.dot(oht, h3, preferred_element_type=jnp.float32)
        c_ref[...] += jnp.sum(oht, axis=1, keepdims=True)

        @pl.when(i == nblk - 1)
        def _():
            emb = s_ref[...] / jnp.maximum(c_ref[...], 1.0)
            emb_ref[...] = emb
            out_ref[...] = (
                jnp.dot(emb, wl_ref[...], preferred_element_type=jnp.float32)
                + bl_ref[...])

    return pl.pallas_call(
        body,
        grid=(nblk,),
        in_specs=[
            pl.BlockSpec((NC, blk, f), lambda i: (0, i, 0)),
            pl.BlockSpec((blk, f), lambda i: (i, 0)),
            pl.BlockSpec((NC, blk, 16), lambda i: (0, i, 0)),
            pl.BlockSpec((1, f), lambda i: (0, 0)),
            pl.BlockSpec((1, 1, blk), lambda i: (i, 0, 0)),
            pl.BlockSpec((f, outdim), lambda i: (0, 0)),
            pl.BlockSpec((1, outdim), lambda i: (0, 0)),
        ],
        out_specs=[
            pl.BlockSpec((g, outdim), lambda i: (0, 0)),
            pl.BlockSpec((g, f), lambda i: (0, 0)),
        ],
        out_shape=[
            jax.ShapeDtypeStruct((g, outdim), jnp.float32),
            jax.ShapeDtypeStruct((g, f), jnp.float32),
        ],
        scratch_shapes=[
            pltpu.VMEM((g, f), jnp.float32),
            pltpu.VMEM((g, 1), jnp.float32),
        ],
    )(agg, zd3, deg, b3, batch3, wl, bl)


def kernel(x, edge_index, batch, W1, b1, W2, b2, W3, b3, Wl, bl):
    n, f = x.shape
    e = edge_index.shape[1]
    g = 128
    blk = 1000
    epw = e // NW
    ch = epw // B_EDGE
    rpt = n // NS

    src3 = edge_index[0].reshape(NW, ch, B_EDGE)
    dst3 = edge_index[1].reshape(NW, ch, B_EDGE)
    batch3 = batch.reshape(n // blk, 1, blk)
    ones_rows = jnp.ones((B_EDGE, 16), jnp.float32)
    zero16 = jnp.zeros((rpt, 16), jnp.float32)
    zero_rows = jnp.zeros((rpt, f), jnp.float32)

    deg = _sc_degree(dst3, ones_rows, zero16)

    zd1 = _tc_layer1(x, W1, deg, blk)
    agg1 = _sc_aggregate(zd1, src3, dst3, zero_rows)
    zd2 = _tc_layer(agg1, zd1, deg, b1.reshape(1, -1), W2, blk)
    agg2 = _sc_aggregate(zd2, src3, dst3, zero_rows)
    zd3 = _tc_layer(agg2, zd2, deg, b2.reshape(1, -1), W3, blk)
    agg3 = _sc_aggregate(zd3, src3, dst3, zero_rows)
    out, emb = _tc_pool(agg3, zd3, deg, b3.reshape(1, -1), batch3,
                        Wl, bl.reshape(1, -1), g, blk)
    return (out, emb)


# speedup vs baseline: 12.9143x; 12.9143x over previous
import functools

import jax
import jax.numpy as jnp
from jax import lax
from jax.experimental import pallas as pl
from jax.experimental.pallas import tpu as pltpu
from jax.experimental.pallas import tpu_sc as plsc

NC = 2
NS = 16
NW = NC * NS

B_EDGE = 80


def _sc_aggregate(zd, src3, dst3, zero_rows):
    n, f = zd.shape
    nw, ch, bb = src3.shape
    rpt = n // NS
    mesh = plsc.VectorSubcoreMesh(core_axis_name="c", subcore_axis_name="s")

    @functools.partial(
        pl.kernel,
        out_type=jax.ShapeDtypeStruct((NC, NS, rpt, f), jnp.float32),
        mesh=mesh,
        scratch_types=[
            pltpu.VMEM((ch, bb), jnp.int32),
            pltpu.VMEM((ch, bb), jnp.int32),
            pltpu.VMEM((bb, f), jnp.float32),
            pltpu.VMEM_SHARED((n, f), jnp.float32),
            pltpu.SemaphoreType.DMA,
        ],
    )
    def k(zd_hbm, src_hbm, dst_hbm, zero_hbm, out_hbm,
          sslab, dslab, rows, acc, sem):
        c = lax.axis_index("c")
        s = lax.axis_index("s")
        w = s * NC + c
        pltpu.sync_copy(src_hbm.at[w], sslab)
        pltpu.sync_copy(dst_hbm.at[w], dslab)
        pltpu.sync_copy(zero_hbm, acc.at[pl.ds(s * rpt, rpt)])
        plsc.subcore_barrier()

        def chunk(j, carry):
            pltpu.async_copy(zd_hbm.at[sslab.at[j]], rows, sem).wait()
            pltpu.sync_copy(rows, acc.at[dslab.at[j]], add=True)
            return carry

        lax.fori_loop(0, ch, chunk, 0)
        plsc.subcore_barrier()
        pltpu.sync_copy(acc.at[pl.ds(s * rpt, rpt)], out_hbm.at[c, s])

    return k(zd, src3, dst3, zero_rows).reshape(NC, n, f)


def _deg_to_d(deg_ref):
    dg = deg_ref[0, :, 0:1] + deg_ref[1, :, 0:1] + 1.0
    return lax.rsqrt(dg)


def _tc_layer1(x, w1, deg, blk):
    n, f = x.shape
    h = w1.shape[1]
    grid = (n // blk,)

    def body(x_ref, w_ref, deg_ref, o_ref):
        d = _deg_to_d(deg_ref)
        z = jnp.dot(x_ref[...], w_ref[...], preferred_element_type=jnp.float32)
        o_ref[...] = z * d

    return pl.pallas_call(
        body,
        grid=grid,
        in_specs=[
            pl.BlockSpec((blk, f), lambda i: (i, 0)),
            pl.BlockSpec((f, h), lambda i: (0, 0)),
            pl.BlockSpec((NC, blk, 128), lambda i: (0, i, 0)),
        ],
        out_specs=pl.BlockSpec((blk, h), lambda i: (i, 0)),
        out_shape=jax.ShapeDtypeStruct((n, h), jnp.float32),
    )(x, w1, deg)


def _tc_layer(agg, zd_prev, deg, b_prev, w, blk):
    n, f = zd_prev.shape
    h = w.shape[1]
    grid = (n // blk,)

    def body(agg_ref, zd_ref, deg_ref, b_ref, w_ref, o_ref):
        d = _deg_to_d(deg_ref)
        pre = d * (agg_ref[0] + agg_ref[1] + zd_ref[...]) + b_ref[...]
        a = jnp.maximum(pre, 0.0)
        z = jnp.dot(a, w_ref[...], preferred_element_type=jnp.float32)
        o_ref[...] = z * d

    return pl.pallas_call(
        body,
        grid=grid,
        in_specs=[
            pl.BlockSpec((NC, blk, f), lambda i: (0, i, 0)),
            pl.BlockSpec((blk, f), lambda i: (i, 0)),
            pl.BlockSpec((NC, blk, 128), lambda i: (0, i, 0)),
            pl.BlockSpec((1, f), lambda i: (0, 0)),
            pl.BlockSpec((f, h), lambda i: (0, 0)),
        ],
        out_specs=pl.BlockSpec((blk, h), lambda i: (i, 0)),
        out_shape=jax.ShapeDtypeStruct((n, h), jnp.float32),
    )(agg, zd_prev, deg, b_prev, w)


def _tc_pool(agg, zd3, deg, b3, batch3, wl, bl, num_graphs, blk):
    n, f = zd3.shape
    outdim = wl.shape[1]
    g = num_graphs
    nblk = n // blk

    def body(agg_ref, zd_ref, deg_ref, b_ref, batch_ref, wl_ref, bl_ref,
             out_ref, emb_ref, s_ref, c_ref):
        i = pl.program_id(0)
        d = _deg_to_d(deg_ref)
        h3 = d * (agg_ref[0] + agg_ref[1] + zd_ref[...]) + b_ref[...]
        bvec = batch_ref[0, 0, :]
        gid = lax.broadcasted_iota(jnp.int32, (g, blk), 0)
        oht = (gid == bvec[None, :]).astype(jnp.float32)

        @pl.when(i == 0)
        def _():
            s_ref[...] = jnp.zeros_like(s_ref)
            c_ref[...] = jnp.zeros_like(c_ref)

        s_ref[...] += jnp.dot(oht, h3, preferred_element_type=jnp.float32)
        c_ref[...] += jnp.sum(oht, axis=1, keepdims=True)

        @pl.when(i == nblk - 1)
        def _():
            emb = s_ref[...] / jnp.maximum(c_ref[...], 1.0)
            emb_ref[...] = emb
            out_ref[...] = (
                jnp.dot(emb, wl_ref[...], preferred_element_type=jnp.float32)
                + bl_ref[...])

    return pl.pallas_call(
        body,
        grid=(nblk,),
        in_specs=[
            pl.BlockSpec((NC, blk, f), lambda i: (0, i, 0)),
            pl.BlockSpec((blk, f), lambda i: (i, 0)),
            pl.BlockSpec((NC, blk, 128), lambda i: (0, i, 0)),
            pl.BlockSpec((1, f), lambda i: (0, 0)),
            pl.BlockSpec((1, 1, blk), lambda i: (i, 0, 0)),
            pl.BlockSpec((f, outdim), lambda i: (0, 0)),
            pl.BlockSpec((1, outdim), lambda i: (0, 0)),
        ],
        out_specs=[
            pl.BlockSpec((g, outdim), lambda i: (0, 0)),
            pl.BlockSpec((g, f), lambda i: (0, 0)),
        ],
        out_shape=[
            jax.ShapeDtypeStruct((g, outdim), jnp.float32),
            jax.ShapeDtypeStruct((g, f), jnp.float32),
        ],
        scratch_shapes=[
            pltpu.VMEM((g, f), jnp.float32),
            pltpu.VMEM((g, 1), jnp.float32),
        ],
    )(agg, zd3, deg, b3, batch3, wl, bl)


def kernel(x, edge_index, batch, W1, b1, W2, b2, W3, b3, Wl, bl):
    n, f = x.shape
    e = edge_index.shape[1]
    g = 128
    blk = 1000
    epw = e // NW
    ch = epw // B_EDGE
    rpt = n // NS

    src3 = edge_index[0].reshape(NW, ch, B_EDGE)
    dst3 = edge_index[1].reshape(NW, ch, B_EDGE)
    batch3 = batch.reshape(n // blk, 1, blk)
    zero_rows = jnp.zeros((rpt, f), jnp.float32)
    ones_nf = jnp.ones((n, f), jnp.float32)

    deg = _sc_aggregate(ones_nf, src3, dst3, zero_rows)

    zd1 = _tc_layer1(x, W1, deg, blk)
    agg1 = _sc_aggregate(zd1, src3, dst3, zero_rows)
    zd2 = _tc_layer(agg1, zd1, deg, b1.reshape(1, -1), W2, blk)
    agg2 = _sc_aggregate(zd2, src3, dst3, zero_rows)
    zd3 = _tc_layer(agg2, zd2, deg, b2.reshape(1, -1), W3, blk)
    agg3 = _sc_aggregate(zd3, src3, dst3, zero_rows)
    out, emb = _tc_pool(agg3, zd3, deg, b3.reshape(1, -1), batch3,
                        Wl, bl.reshape(1, -1), g, blk)
    return (out, emb)


# speedup vs baseline: 19.0833x; 1.4777x over previous
import functools

import jax
import jax.numpy as jnp
from jax import lax
from jax.experimental import pallas as pl
from jax.experimental.pallas import tpu as pltpu
from jax.experimental.pallas import tpu_sc as plsc

NC = 2
NS = 16
NW = NC * NS

B_EDGE = 80


def _sc_aggregate(zd, src4, dst4, zero_rows):
    n, f = zd.shape
    nw, ph, cpp, bb = src4.shape
    assert cpp % 2 == 1
    rpt = n // NS
    mesh = plsc.VectorSubcoreMesh(core_axis_name="c", subcore_axis_name="s")

    @functools.partial(
        pl.kernel,
        out_type=jax.ShapeDtypeStruct((NC, NS, rpt, f), jnp.float32),
        mesh=mesh,
        scratch_types=[
            pltpu.VMEM((cpp, bb), jnp.int32),
            pltpu.VMEM((cpp, bb), jnp.int32),
            pltpu.VMEM((bb, f), jnp.float32),
            pltpu.VMEM((bb, f), jnp.float32),
            pltpu.VMEM_SHARED((n, f), jnp.float32),
            pltpu.SemaphoreType.DMA,
            pltpu.SemaphoreType.DMA,
        ],
    )
    def k(zd_hbm, src_hbm, dst_hbm, zero_hbm, out_hbm,
          sslab, dslab, rows0, rows1, acc, sem0, sem1):
        c = lax.axis_index("c")
        s = lax.axis_index("s")
        w = s * NC + c
        pltpu.sync_copy(zero_hbm, acc.at[pl.ds(s * rpt, rpt)])
        plsc.subcore_barrier()

        def gather(j, buf, sem):
            pltpu.async_copy(zd_hbm.at[sslab.at[j]], buf, sem)

        def wait_scatter(j, buf, sem):
            pltpu.make_async_copy(zd_hbm.at[sslab.at[j]], buf, sem).wait()
            pltpu.sync_copy(buf, acc.at[dslab.at[j]], add=True)

        for p in range(ph):
            pltpu.sync_copy(src_hbm.at[w, p], sslab)
            pltpu.sync_copy(dst_hbm.at[w, p], dslab)
            gather(0, rows0, sem0)

            def two_chunks(i, carry):
                j = 2 * i
                gather(j + 1, rows1, sem1)
                wait_scatter(j, rows0, sem0)
                gather(j + 2, rows0, sem0)
                wait_scatter(j + 1, rows1, sem1)
                return carry

            lax.fori_loop(0, (cpp - 1) // 2, two_chunks, 0)
            wait_scatter(cpp - 1, rows0, sem0)
        plsc.subcore_barrier()
        pltpu.sync_copy(acc.at[pl.ds(s * rpt, rpt)], out_hbm.at[c, s])

    return k(zd, src4, dst4, zero_rows).reshape(NC, n, f)


def _sc_degree(dst_pad, n):
    nw, rws, _ = dst_pad.shape
    rpt = -(-n // (NS * 128)) * 128
    npad = rpt * NS
    rr = rpt // 128
    mesh = plsc.VectorSubcoreMesh(core_axis_name="c", subcore_axis_name="s")

    @functools.partial(
        pl.kernel,
        out_type=jax.ShapeDtypeStruct((NC, NS, rr, 128), jnp.float32),
        mesh=mesh,
        scratch_types=[
            pltpu.VMEM((rws, 128), jnp.int32),
            pltpu.VMEM((NS, rr, 128), jnp.float32),
            pltpu.VMEM((NS,), jnp.int32),
            pltpu.VMEM_SHARED((NS, rr, 128), jnp.float32),
        ],
    )
    def k(dst_hbm, out_hbm, slab, hist, idv, total):
        c = lax.axis_index("c")
        s = lax.axis_index("s")
        w = s * NC + c
        pltpu.sync_copy(dst_hbm.at[w], slab)
        zero16 = jnp.zeros((16,), jnp.float32)
        one16 = jnp.ones((16,), jnp.float32)
        iota16 = lax.iota(jnp.int32, 16)
        idv[...] = iota16

        def zero_hist(t, carry):
            for r in range(rr):
                for cc in range(8):
                    hist[t, r, pl.ds(cc * 16, 16)] = zero16
            return carry

        lax.fori_loop(0, NS, zero_hist, 0)
        pltpu.sync_copy(hist.at[0], total.at[s])

        def count(j, carry):
            for i in range(8):
                idx = slab[j, pl.ds(i * 16, 16)]
                hi = idx // rpt
                rem = idx - hi * rpt
                mid = rem // 128
                lo = rem - mid * 128
                plsc.addupdate_scatter(hist, [hi, mid, lo], one16)
            return carry

        lax.fori_loop(0, rws, count, 0)
        plsc.subcore_barrier()
        pltpu.sync_copy(hist, total.at[idv], add=True)
        plsc.subcore_barrier()
        pltpu.sync_copy(total.at[s], out_hbm.at[c, s])

    return k(dst_pad).reshape(NC, npad)


def _deg_to_d(deg_ref):
    dg = deg_ref[0] + deg_ref[1] + 1.0
    return lax.rsqrt(dg)


def _tc_layer1(x, w1, deg, blk):
    n, f = x.shape
    h = w1.shape[1]
    grid = (n // blk,)

    def body(x_ref, w_ref, deg_ref, o_ref):
        d = _deg_to_d(deg_ref)
        z = jnp.dot(x_ref[...], w_ref[...], preferred_element_type=jnp.float32)
        o_ref[...] = z * d

    return pl.pallas_call(
        body,
        grid=grid,
        in_specs=[
            pl.BlockSpec((blk, f), lambda i: (i, 0)),
            pl.BlockSpec((f, h), lambda i: (0, 0)),
            pl.BlockSpec((NC, blk, 1), lambda i: (0, i, 0)),
        ],
        out_specs=pl.BlockSpec((blk, h), lambda i: (i, 0)),
        out_shape=jax.ShapeDtypeStruct((n, h), jnp.float32),
    )(x, w1, deg)


def _tc_layer(agg, zd_prev, deg, b_prev, w, blk):
    n, f = zd_prev.shape
    h = w.shape[1]
    grid = (n // blk,)

    def body(agg_ref, zd_ref, deg_ref, b_ref, w_ref, o_ref):
        d = _deg_to_d(deg_ref)
        pre = d * (agg_ref[0] + agg_ref[1] + zd_ref[...]) + b_ref[...]
        a = jnp.maximum(pre, 0.0)
        z = jnp.dot(a, w_ref[...], preferred_element_type=jnp.float32)
        o_ref[...] = z * d

    return pl.pallas_call(
        body,
        grid=grid,
        in_specs=[
            pl.BlockSpec((NC, blk, f), lambda i: (0, i, 0)),
            pl.BlockSpec((blk, f), lambda i: (i, 0)),
            pl.BlockSpec((NC, blk, 1), lambda i: (0, i, 0)),
            pl.BlockSpec((1, f), lambda i: (0, 0)),
            pl.BlockSpec((f, h), lambda i: (0, 0)),
        ],
        out_specs=pl.BlockSpec((blk, h), lambda i: (i, 0)),
        out_shape=jax.ShapeDtypeStruct((n, h), jnp.float32),
    )(agg, zd_prev, deg, b_prev, w)


def _tc_pool(agg, zd3, deg, b3, batch3, wl, bl, num_graphs, blk):
    n, f = zd3.shape
    outdim = wl.shape[1]
    g = num_graphs
    nblk = n // blk

    def body(agg_ref, zd_ref, deg_ref, b_ref, batch_ref, wl_ref, bl_ref,
             out_ref, emb_ref, s_ref, c_ref):
        i = pl.program_id(0)
        d = _deg_to_d(deg_ref)
        h3 = d * (agg_ref[0] + agg_ref[1] + zd_ref[...]) + b_ref[...]
        bvec = batch_ref[0, 0, :]
        gid = lax.broadcasted_iota(jnp.int32, (g, blk), 0)
        oht = (gid == bvec[None, :]).astype(jnp.float32)

        @pl.when(i == 0)
        def _():
            s_ref[...] = jnp.zeros_like(s_ref)
            c_ref[...] = jnp.zeros_like(c_ref)

        s_ref[...] += jnp.dot(oht, h3, preferred_element_type=jnp.float32)
        c_ref[...] += jnp.sum(oht, axis=1, keepdims=True)

        @pl.when(i == nblk - 1)
        def _():
            emb = s_ref[...] / jnp.maximum(c_ref[...], 1.0)
            emb_ref[...] = emb
            out_ref[...] = (
                jnp.dot(emb, wl_ref[...], preferred_element_type=jnp.float32)
                + bl_ref[...])

    return pl.pallas_call(
        body,
        grid=(nblk,),
        in_specs=[
            pl.BlockSpec((NC, blk, f), lambda i: (0, i, 0)),
            pl.BlockSpec((blk, f), lambda i: (i, 0)),
            pl.BlockSpec((NC, blk, 1), lambda i: (0, i, 0)),
            pl.BlockSpec((1, f), lambda i: (0, 0)),
            pl.BlockSpec((1, 1, blk), lambda i: (i, 0, 0)),
            pl.BlockSpec((f, outdim), lambda i: (0, 0)),
            pl.BlockSpec((1, outdim), lambda i: (0, 0)),
        ],
        out_specs=[
            pl.BlockSpec((g, outdim), lambda i: (0, 0)),
            pl.BlockSpec((g, f), lambda i: (0, 0)),
        ],
        out_shape=[
            jax.ShapeDtypeStruct((g, outdim), jnp.float32),
            jax.ShapeDtypeStruct((g, f), jnp.float32),
        ],
        scratch_shapes=[
            pltpu.VMEM((g, f), jnp.float32),
            pltpu.VMEM((g, 1), jnp.float32),
        ],
    )(agg, zd3, deg, b3, batch3, wl, bl)


def kernel(x, edge_index, batch, W1, b1, W2, b2, W3, b3, Wl, bl):
    n, f = x.shape
    e = edge_index.shape[1]
    g = 128
    blk = 1000
    epw = e // NW
    ch = epw // B_EDGE
    rpt = n // NS

    cpp = 25
    ph = ch // cpp
    assert ph * cpp == ch
    src3 = edge_index[0].reshape(NW, ph, cpp, B_EDGE)
    dst3 = edge_index[1].reshape(NW, ph, cpp, B_EDGE)
    batch3 = batch.reshape(n // blk, 1, blk)
    zero_rows = jnp.zeros((rpt, f), jnp.float32)

    ones_nf = jnp.ones((n, f), jnp.float32)
    deg = _sc_aggregate(ones_nf, src3, dst3, zero_rows)[:, :, 0:1]

    zd1 = _tc_layer1(x, W1, deg, blk)
    agg1 = _sc_aggregate(zd1, src3, dst3, zero_rows)
    zd2 = _tc_layer(agg1, zd1, deg, b1.reshape(1, -1), W2, blk)
    agg2 = _sc_aggregate(zd2, src3, dst3, zero_rows)
    zd3 = _tc_layer(agg2, zd2, deg, b2.reshape(1, -1), W3, blk)
    agg3 = _sc_aggregate(zd3, src3, dst3, zero_rows)
    out, emb = _tc_pool(agg3, zd3, deg, b3.reshape(1, -1), batch3,
                        Wl, bl.reshape(1, -1), g, blk)
    return (out, emb)


# speedup vs baseline: 21.0139x; 1.1012x over previous
import functools

import jax
import jax.numpy as jnp
from jax import lax
from jax.experimental import pallas as pl
from jax.experimental.pallas import tpu as pltpu
from jax.experimental.pallas import tpu_sc as plsc

NC = 2
NS = 16
NW = NC * NS

B_EDGE = 80


def _sc_aggregate(zd, src4, dst4, zero_rows):
    n, f = zd.shape
    nw, ph, cpp, bb = src4.shape
    assert cpp % 2 == 1
    rpt = n // NS
    mesh = plsc.VectorSubcoreMesh(core_axis_name="c", subcore_axis_name="s")

    @functools.partial(
        pl.kernel,
        out_type=jax.ShapeDtypeStruct((NC, NS, rpt, f), jnp.float32),
        mesh=mesh,
        scratch_types=[
            pltpu.VMEM((cpp, bb), jnp.int32),
            pltpu.VMEM((cpp, bb), jnp.int32),
            pltpu.VMEM((bb, f), jnp.float32),
            pltpu.VMEM((bb, f), jnp.float32),
            pltpu.VMEM_SHARED((n, f), jnp.float32),
            pltpu.SemaphoreType.DMA,
            pltpu.SemaphoreType.DMA,
        ],
    )
    def k(zd_hbm, src_hbm, dst_hbm, zero_hbm, out_hbm,
          sslab, dslab, rows0, rows1, acc, sem0, sem1):
        c = lax.axis_index("c")
        s = lax.axis_index("s")
        w = s * NC + c
        pltpu.sync_copy(zero_hbm, acc.at[pl.ds(s * rpt, rpt)])
        plsc.subcore_barrier()

        def gather(j, buf, sem):
            pltpu.async_copy(zd_hbm.at[sslab.at[j]], buf, sem)

        def wait_scatter(j, buf, sem):
            pltpu.make_async_copy(zd_hbm.at[sslab.at[j]], buf, sem).wait()
            pltpu.sync_copy(buf, acc.at[dslab.at[j]], add=True)

        for p in range(ph):
            pltpu.sync_copy(src_hbm.at[w, p], sslab)
            pltpu.sync_copy(dst_hbm.at[w, p], dslab)
            gather(0, rows0, sem0)

            def two_chunks(i, carry):
                j = 2 * i
                gather(j + 1, rows1, sem1)
                wait_scatter(j, rows0, sem0)
                gather(j + 2, rows0, sem0)
                wait_scatter(j + 1, rows1, sem1)
                return carry

            lax.fori_loop(0, (cpp - 1) // 2, two_chunks, 0)
            wait_scatter(cpp - 1, rows0, sem0)
        plsc.subcore_barrier()
        pltpu.sync_copy(acc.at[pl.ds(s * rpt, rpt)], out_hbm.at[c, s])

    return k(zd, src4, dst4, zero_rows).reshape(NC, n, f)


def _sc_degree(dst4, zero_rows, ones_rows):
    nw, ph, cpp, bb = dst4.shape
    assert cpp % 2 == 1
    rpt, f = zero_rows.shape
    n = rpt * NS
    mesh = plsc.VectorSubcoreMesh(core_axis_name="c", subcore_axis_name="s")

    @functools.partial(
        pl.kernel,
        out_type=jax.ShapeDtypeStruct((NC, NS, rpt, f), jnp.float32),
        mesh=mesh,
        scratch_types=[
            pltpu.VMEM((cpp, bb), jnp.int32),
            pltpu.VMEM((bb, f), jnp.float32),
            pltpu.VMEM_SHARED((n, f), jnp.float32),
            pltpu.SemaphoreType.DMA,
            pltpu.SemaphoreType.DMA,
        ],
    )
    def k(dst_hbm, zero_hbm, ones_hbm, out_hbm, dslab, rows, acc, sem0, sem1):
        c = lax.axis_index("c")
        s = lax.axis_index("s")
        w = s * NC + c
        pltpu.sync_copy(ones_hbm, rows)
        pltpu.sync_copy(zero_hbm, acc.at[pl.ds(s * rpt, rpt)])
        plsc.subcore_barrier()

        def scat(j, sem):
            pltpu.async_copy(rows, acc.at[dslab.at[j]], sem, add=True)

        def wait(j, sem):
            pltpu.make_async_copy(rows, acc.at[dslab.at[j]], sem).wait()

        for p in range(ph):
            pltpu.sync_copy(dst_hbm.at[w, p], dslab)
            scat(0, sem0)

            def two_chunks(i, carry):
                j = 2 * i
                scat(j + 1, sem1)
                wait(j, sem0)
                scat(j + 2, sem0)
                wait(j + 1, sem1)
                return carry

            lax.fori_loop(0, (cpp - 1) // 2, two_chunks, 0)
            wait(cpp - 1, sem0)
        plsc.subcore_barrier()
        pltpu.sync_copy(acc.at[pl.ds(s * rpt, rpt)], out_hbm.at[c, s])

    return k(dst4, zero_rows, ones_rows).reshape(NC, n, f)


def _deg_to_d(deg_ref):
    dg = deg_ref[0] + deg_ref[1] + 1.0
    return lax.rsqrt(dg)


def _tc_layer1(x, w1, deg, blk):
    n, f = x.shape
    h = w1.shape[1]
    grid = (n // blk,)

    def body(x_ref, w_ref, deg_ref, o_ref):
        d = _deg_to_d(deg_ref)
        z = jnp.dot(x_ref[...], w_ref[...], preferred_element_type=jnp.float32)
        o_ref[...] = z * d

    return pl.pallas_call(
        body,
        grid=grid,
        in_specs=[
            pl.BlockSpec((blk, f), lambda i: (i, 0)),
            pl.BlockSpec((f, h), lambda i: (0, 0)),
            pl.BlockSpec((NC, blk, 1), lambda i: (0, i, 0)),
        ],
        out_specs=pl.BlockSpec((blk, h), lambda i: (i, 0)),
        out_shape=jax.ShapeDtypeStruct((n, h), jnp.float32),
    )(x, w1, deg)


def _tc_layer(agg, zd_prev, deg, b_prev, w, blk):
    n, f = zd_prev.shape
    h = w.shape[1]
    grid = (n // blk,)

    def body(agg_ref, zd_ref, deg_ref, b_ref, w_ref, o_ref):
        d = _deg_to_d(deg_ref)
        pre = d * (agg_ref[0] + agg_ref[1] + zd_ref[...]) + b_ref[...]
        a = jnp.maximum(pre, 0.0)
        z = jnp.dot(a, w_ref[...], preferred_element_type=jnp.float32)
        o_ref[...] = z * d

    return pl.pallas_call(
        body,
        grid=grid,
        in_specs=[
            pl.BlockSpec((NC, blk, f), lambda i: (0, i, 0)),
            pl.BlockSpec((blk, f), lambda i: (i, 0)),
            pl.BlockSpec((NC, blk, 1), lambda i: (0, i, 0)),
            pl.BlockSpec((1, f), lambda i: (0, 0)),
            pl.BlockSpec((f, h), lambda i: (0, 0)),
        ],
        out_specs=pl.BlockSpec((blk, h), lambda i: (i, 0)),
        out_shape=jax.ShapeDtypeStruct((n, h), jnp.float32),
    )(agg, zd_prev, deg, b_prev, w)


def _tc_pool(agg, zd3, deg, b3, batch3, wl, bl, num_graphs, blk):
    n, f = zd3.shape
    outdim = wl.shape[1]
    g = num_graphs
    nblk = n // blk

    def body(agg_ref, zd_ref, deg_ref, b_ref, batch_ref, wl_ref, bl_ref,
             out_ref, emb_ref, s_ref, c_ref):
        i = pl.program_id(0)
        d = _deg_to_d(deg_ref)
        h3 = d * (agg_ref[0] + agg_ref[1] + zd_ref[...]) + b_ref[...]
        bvec = batch_ref[0, 0, :]
        gid = lax.broadcasted_iota(jnp.int32, (g, blk), 0)
        oht = (gid == bvec[None, :]).astype(jnp.float32)

        @pl.when(i == 0)
        def _():
            s_ref[...] = jnp.zeros_like(s_ref)
            c_ref[...] = jnp.zeros_like(c_ref)

        s_ref[...] += jnp.dot(oht, h3, preferred_element_type=jnp.float32)
        c_ref[...] += jnp.sum(oht, axis=1, keepdims=True)

        @pl.when(i == nblk - 1)
        def _():
            emb = s_ref[...] / jnp.maximum(c_ref[...], 1.0)
            emb_ref[...] = emb
            out_ref[...] = (
                jnp.dot(emb, wl_ref[...], preferred_element_type=jnp.float32)
                + bl_ref[...])

    return pl.pallas_call(
        body,
        grid=(nblk,),
        in_specs=[
            pl.BlockSpec((NC, blk, f), lambda i: (0, i, 0)),
            pl.BlockSpec((blk, f), lambda i: (i, 0)),
            pl.BlockSpec((NC, blk, 1), lambda i: (0, i, 0)),
            pl.BlockSpec((1, f), lambda i: (0, 0)),
            pl.BlockSpec((1, 1, blk), lambda i: (i, 0, 0)),
            pl.BlockSpec((f, outdim), lambda i: (0, 0)),
            pl.BlockSpec((1, outdim), lambda i: (0, 0)),
        ],
        out_specs=[
            pl.BlockSpec((g, outdim), lambda i: (0, 0)),
            pl.BlockSpec((g, f), lambda i: (0, 0)),
        ],
        out_shape=[
            jax.ShapeDtypeStruct((g, outdim), jnp.float32),
            jax.ShapeDtypeStruct((g, f), jnp.float32),
        ],
        scratch_shapes=[
            pltpu.VMEM((g, f), jnp.float32),
            pltpu.VMEM((g, 1), jnp.float32),
        ],
    )(agg, zd3, deg, b3, batch3, wl, bl)


def kernel(x, edge_index, batch, W1, b1, W2, b2, W3, b3, Wl, bl):
    n, f = x.shape
    e = edge_index.shape[1]
    g = 128
    blk = 1000
    epw = e // NW
    ch = epw // B_EDGE
    rpt = n // NS

    cpp = 25
    ph = ch // cpp
    assert ph * cpp == ch
    src3 = edge_index[0].reshape(NW, ph, cpp, B_EDGE)
    dst3 = edge_index[1].reshape(NW, ph, cpp, B_EDGE)
    batch3 = batch.reshape(n // blk, 1, blk)
    zero_rows = jnp.zeros((rpt, f), jnp.float32)

    ones_rows = jnp.ones((B_EDGE, f), jnp.float32)
    deg = _sc_degree(dst3, zero_rows, ones_rows)[:, :, 0:1]

    zd1 = _tc_layer1(x, W1, deg, blk)
    agg1 = _sc_aggregate(zd1, src3, dst3, zero_rows)
    zd2 = _tc_layer(agg1, zd1, deg, b1.reshape(1, -1), W2, blk)
    agg2 = _sc_aggregate(zd2, src3, dst3, zero_rows)
    zd3 = _tc_layer(agg2, zd2, deg, b2.reshape(1, -1), W3, blk)
    agg3 = _sc_aggregate(zd3, src3, dst3, zero_rows)
    out, emb = _tc_pool(agg3, zd3, deg, b3.reshape(1, -1), batch3,
                        Wl, bl.reshape(1, -1), g, blk)
    return (out, emb)


# speedup vs baseline: 23.2635x; 1.1071x over previous
import functools

import jax
import jax.numpy as jnp
from jax import lax
from jax.experimental import pallas as pl
from jax.experimental.pallas import tpu as pltpu
from jax.experimental.pallas import tpu_sc as plsc

NC = 2
NS = 16
NW = NC * NS

B_EDGE = 80


def _sc_aggregate(zd, src4, dst4, zero_rows):
    n, f = zd.shape
    nw, ph, cpp, bb = src4.shape
    assert (cpp - 1) % 3 == 0
    rpt = n // NS
    mesh = plsc.VectorSubcoreMesh(core_axis_name="c", subcore_axis_name="s")

    @functools.partial(
        pl.kernel,
        out_type=jax.ShapeDtypeStruct((NC, NS, rpt, f), jnp.float32),
        mesh=mesh,
        scratch_types=[
            pltpu.VMEM((cpp, bb), jnp.int32),
            pltpu.VMEM((cpp, bb), jnp.int32),
            pltpu.VMEM((bb, f), jnp.float32),
            pltpu.VMEM((bb, f), jnp.float32),
            pltpu.VMEM((bb, f), jnp.float32),
            pltpu.VMEM_SHARED((n, f), jnp.float32),
            pltpu.SemaphoreType.DMA,
            pltpu.SemaphoreType.DMA,
            pltpu.SemaphoreType.DMA,
        ],
    )
    def k(zd_hbm, src_hbm, dst_hbm, zero_hbm, out_hbm,
          sslab, dslab, rows0, rows1, rows2, acc, sem0, sem1, sem2):
        c = lax.axis_index("c")
        s = lax.axis_index("s")
        w = s * NC + c
        pltpu.sync_copy(zero_hbm, acc.at[pl.ds(s * rpt, rpt)])
        plsc.subcore_barrier()

        def gather(j, buf, sem):
            pltpu.async_copy(zd_hbm.at[sslab.at[j]], buf, sem)

        def wait_scatter(j, buf, sem):
            pltpu.make_async_copy(zd_hbm.at[sslab.at[j]], buf, sem).wait()
            pltpu.sync_copy(buf, acc.at[dslab.at[j]], add=True)

        for p in range(ph):
            pltpu.sync_copy(src_hbm.at[w, p], sslab)
            pltpu.sync_copy(dst_hbm.at[w, p], dslab)
            gather(0, rows0, sem0)
            gather(1, rows1, sem1)

            def three_chunks(i, carry):
                j = 3 * i
                gather(j + 2, rows2, sem2)
                wait_scatter(j, rows0, sem0)
                gather(j + 3, rows0, sem0)
                wait_scatter(j + 1, rows1, sem1)

                @pl.when(j + 4 < cpp)
                def _():
                    gather(j + 4, rows1, sem1)

                wait_scatter(j + 2, rows2, sem2)
                return carry

            lax.fori_loop(0, (cpp - 1) // 3, three_chunks, 0)
            wait_scatter(cpp - 1, rows0, sem0)
        plsc.subcore_barrier()
        pltpu.sync_copy(acc.at[pl.ds(s * rpt, rpt)], out_hbm.at[c, s])

    return k(zd, src4, dst4, zero_rows).reshape(NC, n, f)


def _sc_degree(dst4, zero_rows, ones_rows):
    nw, ph, cpp, bb = dst4.shape
    assert cpp % 2 == 1
    rpt, f = zero_rows.shape
    n = rpt * NS
    mesh = plsc.VectorSubcoreMesh(core_axis_name="c", subcore_axis_name="s")

    @functools.partial(
        pl.kernel,
        out_type=jax.ShapeDtypeStruct((NC, NS, rpt, f), jnp.float32),
        mesh=mesh,
        scratch_types=[
            pltpu.VMEM((cpp, bb), jnp.int32),
            pltpu.VMEM((bb, f), jnp.float32),
            pltpu.VMEM_SHARED((n, f), jnp.float32),
            pltpu.SemaphoreType.DMA,
            pltpu.SemaphoreType.DMA,
        ],
    )
    def k(dst_hbm, zero_hbm, ones_hbm, out_hbm, dslab, rows, acc, sem0, sem1):
        c = lax.axis_index("c")
        s = lax.axis_index("s")
        w = s * NC + c
        pltpu.sync_copy(ones_hbm, rows)
        pltpu.sync_copy(zero_hbm, acc.at[pl.ds(s * rpt, rpt)])
        plsc.subcore_barrier()

        def scat(j, sem):
            pltpu.async_copy(rows, acc.at[dslab.at[j]], sem, add=True)

        def wait(j, sem):
            pltpu.make_async_copy(rows, acc.at[dslab.at[j]], sem).wait()

        for p in range(ph):
            pltpu.sync_copy(dst_hbm.at[w, p], dslab)
            scat(0, sem0)

            def two_chunks(i, carry):
                j = 2 * i
                scat(j + 1, sem1)
                wait(j, sem0)
                scat(j + 2, sem0)
                wait(j + 1, sem1)
                return carry

            lax.fori_loop(0, (cpp - 1) // 2, two_chunks, 0)
            wait(cpp - 1, sem0)
        plsc.subcore_barrier()
        pltpu.sync_copy(acc.at[pl.ds(s * rpt, rpt)], out_hbm.at[c, s])

    return k(dst4, zero_rows, ones_rows).reshape(NC, n, f)


def _deg_to_d(deg_ref):
    dg = deg_ref[0] + deg_ref[1] + 1.0
    return lax.rsqrt(dg)


def _tc_layer1(x, w1, deg, blk):
    n, f = x.shape
    h = w1.shape[1]
    grid = (n // blk,)

    def body(x_ref, w_ref, deg_ref, o_ref):
        d = _deg_to_d(deg_ref)
        z = jnp.dot(x_ref[...], w_ref[...], preferred_element_type=jnp.float32)
        o_ref[...] = z * d

    return pl.pallas_call(
        body,
        grid=grid,
        in_specs=[
            pl.BlockSpec((blk, f), lambda i: (i, 0)),
            pl.BlockSpec((f, h), lambda i: (0, 0)),
            pl.BlockSpec((NC, blk, 1), lambda i: (0, i, 0)),
        ],
        out_specs=pl.BlockSpec((blk, h), lambda i: (i, 0)),
        out_shape=jax.ShapeDtypeStruct((n, h), jnp.float32),
    )(x, w1, deg)


def _tc_layer(agg, zd_prev, deg, b_prev, w, blk):
    n, f = zd_prev.shape
    h = w.shape[1]
    grid = (n // blk,)

    def body(agg_ref, zd_ref, deg_ref, b_ref, w_ref, o_ref):
        d = _deg_to_d(deg_ref)
        pre = d * (agg_ref[0] + agg_ref[1] + zd_ref[...]) + b_ref[...]
        a = jnp.maximum(pre, 0.0)
        z = jnp.dot(a, w_ref[...], preferred_element_type=jnp.float32)
        o_ref[...] = z * d

    return pl.pallas_call(
        body,
        grid=grid,
        in_specs=[
            pl.BlockSpec((NC, blk, f), lambda i: (0, i, 0)),
            pl.BlockSpec((blk, f), lambda i: (i, 0)),
            pl.BlockSpec((NC, blk, 1), lambda i: (0, i, 0)),
            pl.BlockSpec((1, f), lambda i: (0, 0)),
            pl.BlockSpec((f, h), lambda i: (0, 0)),
        ],
        out_specs=pl.BlockSpec((blk, h), lambda i: (i, 0)),
        out_shape=jax.ShapeDtypeStruct((n, h), jnp.float32),
    )(agg, zd_prev, deg, b_prev, w)


def _tc_pool(agg, zd3, deg, b3, batch3, wl, bl, num_graphs, blk):
    n, f = zd3.shape
    outdim = wl.shape[1]
    g = num_graphs
    nblk = n // blk

    def body(agg_ref, zd_ref, deg_ref, b_ref, batch_ref, wl_ref, bl_ref,
             out_ref, emb_ref, s_ref, c_ref):
        i = pl.program_id(0)
        d = _deg_to_d(deg_ref)
        h3 = d * (agg_ref[0] + agg_ref[1] + zd_ref[...]) + b_ref[...]
        bvec = batch_ref[0, 0, :]
        gid = lax.broadcasted_iota(jnp.int32, (g, blk), 0)
        oht = (gid == bvec[None, :]).astype(jnp.float32)

        @pl.when(i == 0)
        def _():
            s_ref[...] = jnp.zeros_like(s_ref)
            c_ref[...] = jnp.zeros_like(c_ref)

        s_ref[...] += jnp.dot(oht, h3, preferred_element_type=jnp.float32)
        c_ref[...] += jnp.sum(oht, axis=1, keepdims=True)

        @pl.when(i == nblk - 1)
        def _():
            emb = s_ref[...] / jnp.maximum(c_ref[...], 1.0)
            emb_ref[...] = emb
            out_ref[...] = (
                jnp.dot(emb, wl_ref[...], preferred_element_type=jnp.float32)
                + bl_ref[...])

    return pl.pallas_call(
        body,
        grid=(nblk,),
        in_specs=[
            pl.BlockSpec((NC, blk, f), lambda i: (0, i, 0)),
            pl.BlockSpec((blk, f), lambda i: (i, 0)),
            pl.BlockSpec((NC, blk, 1), lambda i: (0, i, 0)),
            pl.BlockSpec((1, f), lambda i: (0, 0)),
            pl.BlockSpec((1, 1, blk), lambda i: (i, 0, 0)),
            pl.BlockSpec((f, outdim), lambda i: (0, 0)),
            pl.BlockSpec((1, outdim), lambda i: (0, 0)),
        ],
        out_specs=[
            pl.BlockSpec((g, outdim), lambda i: (0, 0)),
            pl.BlockSpec((g, f), lambda i: (0, 0)),
        ],
        out_shape=[
            jax.ShapeDtypeStruct((g, outdim), jnp.float32),
            jax.ShapeDtypeStruct((g, f), jnp.float32),
        ],
        scratch_shapes=[
            pltpu.VMEM((g, f), jnp.float32),
            pltpu.VMEM((g, 1), jnp.float32),
        ],
    )(agg, zd3, deg, b3, batch3, wl, bl)


def kernel(x, edge_index, batch, W1, b1, W2, b2, W3, b3, Wl, bl):
    n, f = x.shape
    e = edge_index.shape[1]
    g = 128
    blk = 1000
    epw = e // NW
    ch = epw // B_EDGE
    rpt = n // NS

    cpp = 25
    ph = ch // cpp
    assert ph * cpp == ch
    src3 = edge_index[0].reshape(NW, ph, cpp, B_EDGE)
    dst3 = edge_index[1].reshape(NW, ph, cpp, B_EDGE)
    batch3 = batch.reshape(n // blk, 1, blk)
    zero_rows = jnp.zeros((rpt, f), jnp.float32)

    ones_rows = jnp.ones((B_EDGE, f), jnp.float32)
    deg = _sc_degree(dst3, zero_rows, ones_rows)[:, :, 0:1]

    zd1 = _tc_layer1(x, W1, deg, blk)
    agg1 = _sc_aggregate(zd1, src3, dst3, zero_rows)
    zd2 = _tc_layer(agg1, zd1, deg, b1.reshape(1, -1), W2, blk)
    agg2 = _sc_aggregate(zd2, src3, dst3, zero_rows)
    zd3 = _tc_layer(agg2, zd2, deg, b2.reshape(1, -1), W3, blk)
    agg3 = _sc_aggregate(zd3, src3, dst3, zero_rows)
    out, emb = _tc_pool(agg3, zd3, deg, b3.reshape(1, -1), batch3,
                        Wl, bl.reshape(1, -1), g, blk)
    return (out, emb)


# speedup vs baseline: 23.6501x; 1.0166x over previous
import functools

import jax
import jax.numpy as jnp
from jax import lax
from jax.experimental import pallas as pl
from jax.experimental.pallas import tpu as pltpu
from jax.experimental.pallas import tpu_sc as plsc

NC = 2
NS = 16
NW = NC * NS

B_EDGE = 80


def _sc_aggregate(zd, src4, dst4, zero_rows):
    n, f = zd.shape
    nw, ph, cpp, bb = src4.shape
    assert (cpp - 1) % 3 == 0
    rpt = n // NS
    mesh = plsc.VectorSubcoreMesh(core_axis_name="c", subcore_axis_name="s")

    @functools.partial(
        pl.kernel,
        out_type=jax.ShapeDtypeStruct((NC, NS, rpt, f), jnp.float32),
        mesh=mesh,
        scratch_types=[
            pltpu.VMEM((cpp, bb), jnp.int32),
            pltpu.VMEM((cpp, bb), jnp.int32),
            pltpu.VMEM((bb, f), jnp.float32),
            pltpu.VMEM((bb, f), jnp.float32),
            pltpu.VMEM((bb, f), jnp.float32),
            pltpu.VMEM_SHARED((n, f), jnp.float32),
            pltpu.SemaphoreType.DMA,
            pltpu.SemaphoreType.DMA,
            pltpu.SemaphoreType.DMA,
        ],
    )
    def k(zd_hbm, src_hbm, dst_hbm, zero_hbm, out_hbm,
          sslab, dslab, rows0, rows1, rows2, acc, sem0, sem1, sem2):
        c = lax.axis_index("c")
        s = lax.axis_index("s")
        w = s * NC + c
        pltpu.sync_copy(zero_hbm, acc.at[pl.ds(s * rpt, rpt)])
        plsc.subcore_barrier()

        def gather(j, buf, sem):
            pltpu.async_copy(zd_hbm.at[sslab.at[j]], buf, sem)

        def wait_scatter(j, buf, sem):
            pltpu.make_async_copy(zd_hbm.at[sslab.at[j]], buf, sem).wait()
            pltpu.sync_copy(buf, acc.at[dslab.at[j]], add=True)

        for p in range(ph):
            pltpu.sync_copy(src_hbm.at[w, p], sslab)
            pltpu.sync_copy(dst_hbm.at[w, p], dslab)
            gather(0, rows0, sem0)
            gather(1, rows1, sem1)

            def three_chunks(i, carry):
                j = 3 * i
                gather(j + 2, rows2, sem2)
                wait_scatter(j, rows0, sem0)
                gather(j + 3, rows0, sem0)
                wait_scatter(j + 1, rows1, sem1)

                @pl.when(j + 4 < cpp)
                def _():
                    gather(j + 4, rows1, sem1)

                wait_scatter(j + 2, rows2, sem2)
                return carry

            lax.fori_loop(0, (cpp - 1) // 3, three_chunks, 0)
            wait_scatter(cpp - 1, rows0, sem0)
        plsc.subcore_barrier()
        pltpu.sync_copy(acc.at[pl.ds(s * rpt, rpt)], out_hbm.at[c, s])

    return k(zd, src4, dst4, zero_rows).reshape(NC, n, f)


def _sc_degree(dst4, zero_rows, ones_rows):
    nw, ph, cpp, bb = dst4.shape
    assert cpp % 2 == 1
    rpt, f = zero_rows.shape
    dt = zero_rows.dtype
    n = rpt * NS
    mesh = plsc.VectorSubcoreMesh(core_axis_name="c", subcore_axis_name="s")

    @functools.partial(
        pl.kernel,
        out_type=jax.ShapeDtypeStruct((NC, NS, rpt, f), dt),
        mesh=mesh,
        scratch_types=[
            pltpu.VMEM((cpp, bb), jnp.int32),
            pltpu.VMEM((bb, f), dt),
            pltpu.VMEM_SHARED((n, f), dt),
            pltpu.SemaphoreType.DMA,
            pltpu.SemaphoreType.DMA,
        ],
    )
    def k(dst_hbm, zero_hbm, ones_hbm, out_hbm, dslab, rows, acc, sem0, sem1):
        c = lax.axis_index("c")
        s = lax.axis_index("s")
        w = s * NC + c
        pltpu.sync_copy(ones_hbm, rows)
        pltpu.sync_copy(zero_hbm, acc.at[pl.ds(s * rpt, rpt)])
        plsc.subcore_barrier()

        def scat(j, sem):
            pltpu.async_copy(rows, acc.at[dslab.at[j]], sem, add=True)

        def wait(j, sem):
            pltpu.make_async_copy(rows, acc.at[dslab.at[j]], sem).wait()

        for p in range(ph):
            pltpu.sync_copy(dst_hbm.at[w, p], dslab)
            scat(0, sem0)

            def two_chunks(i, carry):
                j = 2 * i
                scat(j + 1, sem1)
                wait(j, sem0)
                scat(j + 2, sem0)
                wait(j + 1, sem1)
                return carry

            lax.fori_loop(0, (cpp - 1) // 2, two_chunks, 0)
            wait(cpp - 1, sem0)
        plsc.subcore_barrier()
        pltpu.sync_copy(acc.at[pl.ds(s * rpt, rpt)], out_hbm.at[c, s])

    return k(dst4, zero_rows, ones_rows).reshape(NC, n, f)


def _deg_to_d(deg_ref):
    dg = deg_ref[0] + deg_ref[1] + 1.0
    return lax.rsqrt(dg)


def _tc_layer1(x, w1, deg, blk):
    n, f = x.shape
    h = w1.shape[1]
    grid = (n // blk,)

    def body(x_ref, w_ref, deg_ref, o_ref):
        d = _deg_to_d(deg_ref)
        z = jnp.dot(x_ref[...], w_ref[...], preferred_element_type=jnp.float32)
        o_ref[...] = z * d

    return pl.pallas_call(
        body,
        grid=grid,
        in_specs=[
            pl.BlockSpec((blk, f), lambda i: (i, 0)),
            pl.BlockSpec((f, h), lambda i: (0, 0)),
            pl.BlockSpec((NC, blk, 1), lambda i: (0, i, 0)),
        ],
        out_specs=pl.BlockSpec((blk, h), lambda i: (i, 0)),
        out_shape=jax.ShapeDtypeStruct((n, h), jnp.float32),
    )(x, w1, deg)


def _tc_layer(agg, zd_prev, deg, b_prev, w, blk):
    n, f = zd_prev.shape
    h = w.shape[1]
    grid = (n // blk,)

    def body(agg_ref, zd_ref, deg_ref, b_ref, w_ref, o_ref):
        d = _deg_to_d(deg_ref)
        pre = d * (agg_ref[0] + agg_ref[1] + zd_ref[...]) + b_ref[...]
        a = jnp.maximum(pre, 0.0)
        z = jnp.dot(a, w_ref[...], preferred_element_type=jnp.float32)
        o_ref[...] = z * d

    return pl.pallas_call(
        body,
        grid=grid,
        in_specs=[
            pl.BlockSpec((NC, blk, f), lambda i: (0, i, 0)),
            pl.BlockSpec((blk, f), lambda i: (i, 0)),
            pl.BlockSpec((NC, blk, 1), lambda i: (0, i, 0)),
            pl.BlockSpec((1, f), lambda i: (0, 0)),
            pl.BlockSpec((f, h), lambda i: (0, 0)),
        ],
        out_specs=pl.BlockSpec((blk, h), lambda i: (i, 0)),
        out_shape=jax.ShapeDtypeStruct((n, h), jnp.float32),
    )(agg, zd_prev, deg, b_prev, w)


def _tc_pool(agg, zd3, deg, b3, batch3, wl, bl, num_graphs, blk):
    n, f = zd3.shape
    outdim = wl.shape[1]
    g = num_graphs
    nblk = n // blk

    def body(agg_ref, zd_ref, deg_ref, b_ref, batch_ref, wl_ref, bl_ref,
             out_ref, emb_ref, s_ref, c_ref):
        i = pl.program_id(0)
        d = _deg_to_d(deg_ref)
        h3 = d * (agg_ref[0] + agg_ref[1] + zd_ref[...]) + b_ref[...]
        bvec = batch_ref[0, 0, :]
        gid = lax.broadcasted_iota(jnp.int32, (g, blk), 0)
        oht = (gid == bvec[None, :]).astype(jnp.float32)

        @pl.when(i == 0)
        def _():
            s_ref[...] = jnp.zeros_like(s_ref)
            c_ref[...] = jnp.zeros_like(c_ref)

        s_ref[...] += jnp.dot(oht, h3, preferred_element_type=jnp.float32)
        c_ref[...] += jnp.sum(oht, axis=1, keepdims=True)

        @pl.when(i == nblk - 1)
        def _():
            emb = s_ref[...] / jnp.maximum(c_ref[...], 1.0)
            emb_ref[...] = emb
            out_ref[...] = (
                jnp.dot(emb, wl_ref[...], preferred_element_type=jnp.float32)
                + bl_ref[...])

    return pl.pallas_call(
        body,
        grid=(nblk,),
        in_specs=[
            pl.BlockSpec((NC, blk, f), lambda i: (0, i, 0)),
            pl.BlockSpec((blk, f), lambda i: (i, 0)),
            pl.BlockSpec((NC, blk, 1), lambda i: (0, i, 0)),
            pl.BlockSpec((1, f), lambda i: (0, 0)),
            pl.BlockSpec((1, 1, blk), lambda i: (i, 0, 0)),
            pl.BlockSpec((f, outdim), lambda i: (0, 0)),
            pl.BlockSpec((1, outdim), lambda i: (0, 0)),
        ],
        out_specs=[
            pl.BlockSpec((g, outdim), lambda i: (0, 0)),
            pl.BlockSpec((g, f), lambda i: (0, 0)),
        ],
        out_shape=[
            jax.ShapeDtypeStruct((g, outdim), jnp.float32),
            jax.ShapeDtypeStruct((g, f), jnp.float32),
        ],
        scratch_shapes=[
            pltpu.VMEM((g, f), jnp.float32),
            pltpu.VMEM((g, 1), jnp.float32),
        ],
    )(agg, zd3, deg, b3, batch3, wl, bl)


def kernel(x, edge_index, batch, W1, b1, W2, b2, W3, b3, Wl, bl):
    n, f = x.shape
    e = edge_index.shape[1]
    g = 128
    blk = 2000
    epw = e // NW
    ch = epw // B_EDGE
    rpt = n // NS

    cpp = 25
    ph = ch // cpp
    assert ph * cpp == ch
    src3 = edge_index[0].reshape(NW, ph, cpp, B_EDGE)
    dst3 = edge_index[1].reshape(NW, ph, cpp, B_EDGE)
    batch3 = batch.reshape(n // blk, 1, blk)
    zero_rows = jnp.zeros((rpt, f), jnp.float32)

    ones_rows = jnp.ones((B_EDGE, f), jnp.float32)
    deg = _sc_degree(dst3, zero_rows, ones_rows)[:, :, 0:1]

    zd1 = _tc_layer1(x, W1, deg, blk)
    agg1 = _sc_aggregate(zd1, src3, dst3, zero_rows)
    zd2 = _tc_layer(agg1, zd1, deg, b1.reshape(1, -1), W2, blk)
    agg2 = _sc_aggregate(zd2, src3, dst3, zero_rows)
    zd3 = _tc_layer(agg2, zd2, deg, b2.reshape(1, -1), W3, blk)
    agg3 = _sc_aggregate(zd3, src3, dst3, zero_rows)
    out, emb = _tc_pool(agg3, zd3, deg, b3.reshape(1, -1), batch3,
                        Wl, bl.reshape(1, -1), g, blk)
    return (out, emb)
